# baseline (speedup 1.0000x reference)
import jax
import jax.numpy as jnp
from jax.experimental import pallas as pl
from jax.experimental.pallas import tpu as pltpu

_CHUNKS = 8


def _make_body(batch, n_rows):
    rows_c = n_rows // _CHUNKS

    def body(table_hbm, out_hbm, vmem, sem_rd, sem_wr):
        reads = [
            pltpu.make_async_copy(
                table_hbm.at[pl.ds(k * rows_c, rows_c)],
                vmem.at[pl.ds(k * rows_c, rows_c)],
                sem_rd.at[k],
            )
            for k in range(_CHUNKS)
        ]
        for r in reads:
            r.start()
        for k in range(_CHUNKS):
            reads[k].wait()
            for b in range(batch):
                pltpu.make_async_copy(
                    vmem.at[pl.ds(k * rows_c, rows_c)],
                    out_hbm.at[b].at[pl.ds(k * rows_c, rows_c)],
                    sem_wr,
                ).start()
        pltpu.make_async_copy(out_hbm, out_hbm, sem_wr).wait()

    return body


def kernel(x, table):
    batch = x.shape[0]
    n_rows, embed = table.shape
    return pl.pallas_call(
        _make_body(batch, n_rows),
        in_specs=[pl.BlockSpec(memory_space=pl.ANY)],
        out_specs=pl.BlockSpec(memory_space=pl.ANY),
        out_shape=jax.ShapeDtypeStruct((batch, n_rows, embed), table.dtype),
        scratch_shapes=[
            pltpu.VMEM((n_rows, embed), table.dtype),
            pltpu.SemaphoreType.DMA((_CHUNKS,)),
            pltpu.SemaphoreType.DMA,
        ],
    )(table)


# speedup vs baseline: 1.0088x; 1.0088x over previous
import jax
import jax.numpy as jnp
from jax.experimental import pallas as pl
from jax.experimental.pallas import tpu as pltpu

_CHUNKS = 32


def _make_body(batch, n_rows):
    rows_c = n_rows // _CHUNKS

    def body(table_hbm, out_hbm, vmem, sem_rd, sem_wr):
        reads = [
            pltpu.make_async_copy(
                table_hbm.at[pl.ds(k * rows_c, rows_c)],
                vmem.at[pl.ds(k * rows_c, rows_c)],
                sem_rd.at[k],
            )
            for k in range(_CHUNKS)
        ]
        for r in reads:
            r.start()
        for k in range(_CHUNKS):
            reads[k].wait()
            for b in range(batch):
                pltpu.make_async_copy(
                    vmem.at[pl.ds(k * rows_c, rows_c)],
                    out_hbm.at[b].at[pl.ds(k * rows_c, rows_c)],
                    sem_wr,
                ).start()
        pltpu.make_async_copy(out_hbm, out_hbm, sem_wr).wait()

    return body


def kernel(x, table):
    batch = x.shape[0]
    n_rows, embed = table.shape
    return pl.pallas_call(
        _make_body(batch, n_rows),
        in_specs=[pl.BlockSpec(memory_space=pl.ANY)],
        out_specs=pl.BlockSpec(memory_space=pl.ANY),
        out_shape=jax.ShapeDtypeStruct((batch, n_rows, embed), table.dtype),
        scratch_shapes=[
            pltpu.VMEM((n_rows, embed), table.dtype),
            pltpu.SemaphoreType.DMA((_CHUNKS,)),
            pltpu.SemaphoreType.DMA,
        ],
    )(table)


# speedup vs baseline: 1.0267x; 1.0178x over previous
import jax
import jax.numpy as jnp
from jax.experimental import pallas as pl
from jax.experimental.pallas import tpu as pltpu


def _chunk_sizes(n_rows):
    sizes = [64, 64, 64, 64]
    rem = n_rows - sum(sizes)
    sizes += [128] * (rem // 128)
    assert sum(sizes) == n_rows
    return sizes


def _make_body(batch, n_rows):
    sizes = _chunk_sizes(n_rows)
    offs = [sum(sizes[:k]) for k in range(len(sizes))]
    n_chunks = len(sizes)

    def body(table_hbm, out_hbm, vmem, sem_rd, sem_wr):
        reads = [
            pltpu.make_async_copy(
                table_hbm.at[pl.ds(offs[k], sizes[k])],
                vmem.at[pl.ds(offs[k], sizes[k])],
                sem_rd.at[k],
            )
            for k in range(n_chunks)
        ]
        for r in reads:
            r.start()
        for k in range(n_chunks):
            reads[k].wait()
            for b in range(batch):
                pltpu.make_async_copy(
                    vmem.at[pl.ds(offs[k], sizes[k])],
                    out_hbm.at[b].at[pl.ds(offs[k], sizes[k])],
                    sem_wr,
                ).start()
        pltpu.make_async_copy(out_hbm, out_hbm, sem_wr).wait()

    return body


def kernel(x, table):
    batch = x.shape[0]
    n_rows, embed = table.shape
    n_chunks = len(_chunk_sizes(n_rows))
    return pl.pallas_call(
        _make_body(batch, n_rows),
        in_specs=[pl.BlockSpec(memory_space=pl.ANY)],
        out_specs=pl.BlockSpec(memory_space=pl.ANY),
        out_shape=jax.ShapeDtypeStruct((batch, n_rows, embed), table.dtype),
        scratch_shapes=[
            pltpu.VMEM((n_rows, embed), table.dtype),
            pltpu.SemaphoreType.DMA((n_chunks,)),
            pltpu.SemaphoreType.DMA,
        ],
    )(table)
